# Initial kernel scaffold; baseline (speedup 1.0000x reference)
import jax
import jax.numpy as jnp
from jax.experimental import pallas as pl


def kernel(x, points, indices, W1, b1, g1, be1, W2, b2, g2, be2):
    raise NotImplementedError("write your pallas kernel here")


# speedup vs baseline: 11.8802x; 11.8802x over previous
import functools

import jax
import jax.numpy as jnp
from jax.experimental import pallas as pl
from jax.experimental.pallas import tpu as pltpu
from jax.experimental.pallas import tpu_sc as plsc

_PTS_PAD = 128
_P = 8
_TN = 200
_GW = 128


def _sc_gather(xt, pt, idx):
    M = idx.shape[1]
    C = xt.shape[1]
    mesh = plsc.VectorSubcoreMesh(core_axis_name="c", subcore_axis_name="s")

    @functools.partial(
        pl.kernel,
        out_type=(jax.ShapeDtypeStruct((M, C), xt.dtype),
                  jax.ShapeDtypeStruct((M, _PTS_PAD), pt.dtype)),
        mesh=mesh,
    )
    def k(x_hbm, p_hbm, i_hbm, ox_hbm, op_hbm):
        def body(i_vmem, ox_vmem, op_vmem):
            pltpu.sync_copy(x_hbm.at[i_vmem.at[0]], ox_vmem)
            pltpu.sync_copy(p_hbm.at[i_vmem.at[0]], op_vmem)

        pltpu.emit_pipeline(
            body,
            grid=(M // _GW,),
            in_specs=[pl.BlockSpec((1, _GW), lambda i: (0, i))],
            out_specs=[pl.BlockSpec((_GW, C), lambda i: (i, 0)),
                       pl.BlockSpec((_GW, _PTS_PAD), lambda i: (i, 0))],
            core_axis_name=("c", "s"),
            dimension_semantics=(pltpu.PARALLEL,),
        )(i_hbm, ox_hbm, op_hbm)

    return k(xt, pt, idx)


def _bn_relu(y, g_ref, be_ref):
    cnt = y.shape[0] * y.shape[2]
    s = jnp.sum(y, axis=2, keepdims=True)
    ss = jnp.sum(y * y, axis=2, keepdims=True)
    mean = jnp.sum(s, axis=0, keepdims=True) / cnt
    ex2 = jnp.sum(ss, axis=0, keepdims=True) / cnt
    var = ex2 - mean * mean
    inv = jax.lax.rsqrt(var + 1e-5)
    g = g_ref[...][None]
    be = be_ref[...][None]
    return jnp.maximum((y - mean) * (inv * g) + be, 0.0)


def _mlp_body(pts_ref, nbr_ref, feat_ref, g1_ref, be1_ref, g2_ref, be2_ref,
              w1c_ref, w1p_ref, b1_ref, w2_ref, b2_ref, out_ref):
    B, TN, _ = pts_ref.shape
    HID = w1p_ref.shape[1]
    OUTD = w2_ref.shape[1]
    C = out_ref.shape[2]
    K = OUTD // C

    p = pts_ref[...].reshape(B * TN, _P)
    a = nbr_ref[...].reshape(B * TN, w1p_ref.shape[0])
    y = (jnp.dot(p, w1c_ref[...], preferred_element_type=jnp.float32)
         - jnp.dot(a, w1p_ref[...], preferred_element_type=jnp.float32)
         + b1_ref[...])
    y = _bn_relu(y.reshape(B, TN, HID), g1_ref, be1_ref)
    z = (jnp.dot(y.reshape(B * TN, HID), w2_ref[...],
                 preferred_element_type=jnp.float32)
         + b2_ref[...])
    z = _bn_relu(z.reshape(B, TN, OUTD), g2_ref, be2_ref)

    f = feat_ref[...]
    acc = z[:, :, 0:C] * f[:, :, 0:C]
    for k in range(1, K):
        acc = acc + z[:, :, k * C:(k + 1) * C] * f[:, :, k * C:(k + 1) * C]
    out_ref[...] = acc


def _mlp_call(pts_p, nbr, feats, g1, be1, g2, be2, w1c, w1p, b1, w2t, b2):
    B, N, _ = pts_p.shape
    KP = nbr.shape[2]
    KC = feats.shape[2]
    HID = w2t.shape[0]
    OUTD = w2t.shape[1]
    C = KC // (KP // _PTS_PAD)
    grid = (N // _TN,)
    return pl.pallas_call(
        _mlp_body,
        grid=grid,
        in_specs=[
            pl.BlockSpec((B, _TN, _P), lambda i: (0, i, 0)),
            pl.BlockSpec((B, _TN, KP), lambda i: (0, i, 0)),
            pl.BlockSpec((B, _TN, KC), lambda i: (0, i, 0)),
            pl.BlockSpec((_TN, 1), lambda i: (i, 0)),
            pl.BlockSpec((_TN, 1), lambda i: (i, 0)),
            pl.BlockSpec((_TN, 1), lambda i: (i, 0)),
            pl.BlockSpec((_TN, 1), lambda i: (i, 0)),
            pl.BlockSpec((_P, HID), lambda i: (0, 0)),
            pl.BlockSpec((KP, HID), lambda i: (0, 0)),
            pl.BlockSpec((1, HID), lambda i: (0, 0)),
            pl.BlockSpec((HID, OUTD), lambda i: (0, 0)),
            pl.BlockSpec((1, OUTD), lambda i: (0, 0)),
        ],
        out_specs=pl.BlockSpec((B, _TN, C), lambda i: (0, i, 0)),
        out_shape=jax.ShapeDtypeStruct((B, N, C), jnp.float32),
    )(pts_p, nbr, feats, g1, be1, g2, be2, w1c, w1p, b1, w2t, b2)


def kernel(x, points, indices, W1, b1, g1, be1, W2, b2, g2, be2):
    B, N, C = x.shape
    K = indices.shape[2]
    HID = W1.shape[0]
    OUTD = W2.shape[0]
    KP = K * _PTS_PAD

    xt = x.reshape(B * N, C)
    pt = jnp.pad(points, ((0, 0), (0, 0), (0, _PTS_PAD - 3))).reshape(
        B * N, _PTS_PAD)
    idx = (indices.astype(jnp.int32)
           + (jnp.arange(B, dtype=jnp.int32) * N)[:, None, None])
    idx = idx.reshape(1, B * N * K)
    feats, nbrp = _sc_gather(xt, pt, idx)
    feats = feats.reshape(B, N, K * C)
    nbr = nbrp.reshape(B, N, KP)

    pts_p = jnp.pad(points, ((0, 0), (0, 0), (0, _P - 3)))
    w1_khj = W1.reshape(HID, K, 3)
    w1p = jnp.zeros((HID, K, _PTS_PAD), W1.dtype).at[:, :, :3].set(
        w1_khj).reshape(HID, KP).T
    w1c = jnp.zeros((_P, HID), W1.dtype).at[:3, :].set(
        jnp.sum(w1_khj, axis=1).T)
    w2t = W2.T

    out = _mlp_call(pts_p, nbr, feats,
                    g1.reshape(N, 1), be1.reshape(N, 1),
                    g2.reshape(N, 1), be2.reshape(N, 1),
                    w1c, w1p, b1.reshape(1, HID), w2t, b2.reshape(1, OUTD))
    return (out, points, indices)


# speedup vs baseline: 18.3455x; 1.5442x over previous
import dataclasses
import functools

import jax
import jax.numpy as jnp
from jax.experimental import pallas as pl
from jax.experimental.pallas import tpu as pltpu
from jax.experimental.pallas import tpu_sc as plsc

_P = 8
_TN = 200
_GW = 128
_NW = 32
_VEC = 16


def _sc_gather_feats(xt, idx):
    M = idx.shape[1]
    C = xt.shape[1]
    mesh = plsc.VectorSubcoreMesh(core_axis_name="c", subcore_axis_name="s")

    @functools.partial(
        pl.kernel,
        out_type=jax.ShapeDtypeStruct((M, C), xt.dtype),
        mesh=mesh,
    )
    def k(x_hbm, i_hbm, o_hbm):
        def body(i_vmem, o_vmem):
            pltpu.sync_copy(x_hbm.at[i_vmem.at[0]], o_vmem)

        pltpu.emit_pipeline(
            body,
            grid=(M // _GW,),
            in_specs=[pl.BlockSpec((1, _GW), lambda i: (0, i))],
            out_specs=[pl.BlockSpec((_GW, C), lambda i: (i, 0))],
            core_axis_name=("c", "s"),
            dimension_semantics=(pltpu.PARALLEL,),
        )(i_hbm, o_hbm)

    return k(xt, idx)


def _sc_gather_coords(px, py, pz, idx):
    R = px.shape[0]
    M = idx.shape[0]
    per_w = M // _NW
    CH = 2000
    mesh = plsc.VectorSubcoreMesh(core_axis_name="c", subcore_axis_name="s")
    cp = pltpu.CompilerParams()
    if "needs_layout_passes" in pltpu.CompilerParams.__dataclass_fields__:
        cp = dataclasses.replace(cp, needs_layout_passes=False)

    @functools.partial(
        pl.kernel,
        out_type=tuple(jax.ShapeDtypeStruct((M,), jnp.float32)
                       for _ in range(3)),
        mesh=mesh,
        compiler_params=cp,
        scratch_types=[
            pltpu.VMEM((R,), jnp.float32),
            pltpu.VMEM((R,), jnp.float32),
            pltpu.VMEM((R,), jnp.float32),
            pltpu.VMEM((CH,), jnp.int32),
            pltpu.VMEM((CH,), jnp.float32),
            pltpu.VMEM((CH,), jnp.float32),
            pltpu.VMEM((CH,), jnp.float32),
        ],
    )
    def k(px_hbm, py_hbm, pz_hbm, i_hbm, o0_hbm, o1_hbm, o2_hbm,
          tx_v, ty_v, tz_v, i_v, o0_v, o1_v, o2_v):
        wid = jax.lax.axis_index("s") * 2 + jax.lax.axis_index("c")
        base = wid * per_w
        pltpu.sync_copy(px_hbm, tx_v)
        pltpu.sync_copy(py_hbm, ty_v)
        pltpu.sync_copy(pz_hbm, tz_v)

        @pl.loop(0, per_w, step=CH)
        def _chunk(c0):
            pltpu.sync_copy(i_hbm.at[pl.ds(base + c0, CH)], i_v)

            @pl.loop(0, CH, step=_VEC)
            def _vec(t):
                iv = i_v[pl.ds(t, _VEC)]
                o0_v[pl.ds(t, _VEC)] = plsc.load_gather(tx_v, [iv])
                o1_v[pl.ds(t, _VEC)] = plsc.load_gather(ty_v, [iv])
                o2_v[pl.ds(t, _VEC)] = plsc.load_gather(tz_v, [iv])

            pltpu.sync_copy(o0_v, o0_hbm.at[pl.ds(base + c0, CH)])
            pltpu.sync_copy(o1_v, o1_hbm.at[pl.ds(base + c0, CH)])
            pltpu.sync_copy(o2_v, o2_hbm.at[pl.ds(base + c0, CH)])

    return k(px, py, pz, idx)


def _bn_relu(y, g_ref, be_ref):
    cnt = y.shape[0] * y.shape[2]
    s = jnp.sum(y, axis=2, keepdims=True)
    ss = jnp.sum(y * y, axis=2, keepdims=True)
    mean = jnp.sum(s, axis=0, keepdims=True) / cnt
    ex2 = jnp.sum(ss, axis=0, keepdims=True) / cnt
    var = ex2 - mean * mean
    inv = jax.lax.rsqrt(var + 1e-5)
    g = g_ref[...][None]
    be = be_ref[...][None]
    return jnp.maximum((y - mean) * (inv * g) + be, 0.0)


def _mlp_body(pts_ref, n0_ref, n1_ref, n2_ref, feat_ref,
              g1_ref, be1_ref, g2_ref, be2_ref,
              w1c_ref, w10_ref, w11_ref, w12_ref, b1_ref, w2_ref, b2_ref,
              out_ref):
    B, TN, _ = pts_ref.shape
    HID = w1c_ref.shape[1]
    OUTD = w2_ref.shape[1]
    C = out_ref.shape[2]
    K = OUTD // C

    p = pts_ref[...].reshape(B * TN, _P)
    dot = functools.partial(jnp.dot, preferred_element_type=jnp.float32)
    y = (dot(p, w1c_ref[...])
         - dot(n0_ref[...].reshape(B * TN, K), w10_ref[...])
         - dot(n1_ref[...].reshape(B * TN, K), w11_ref[...])
         - dot(n2_ref[...].reshape(B * TN, K), w12_ref[...])
         + b1_ref[...])
    y = _bn_relu(y.reshape(B, TN, HID), g1_ref, be1_ref)
    z = dot(y.reshape(B * TN, HID), w2_ref[...]) + b2_ref[...]
    z = _bn_relu(z.reshape(B, TN, OUTD), g2_ref, be2_ref)

    f = feat_ref[...]
    acc = z[:, :, 0:C] * f[:, :, 0:C]
    for k in range(1, K):
        acc = acc + z[:, :, k * C:(k + 1) * C] * f[:, :, k * C:(k + 1) * C]
    out_ref[...] = acc


def _mlp_call(pts_p, nbr0, nbr1, nbr2, feats, g1, be1, g2, be2,
              w1c, w10, w11, w12, b1, w2t, b2):
    B, N, _ = pts_p.shape
    K = nbr0.shape[2]
    KC = feats.shape[2]
    HID = w2t.shape[0]
    OUTD = w2t.shape[1]
    C = KC // K
    grid = (N // _TN,)
    nspec = pl.BlockSpec((B, _TN, K), lambda i: (0, i, 0))
    gspec = pl.BlockSpec((_TN, 1), lambda i: (i, 0))
    return pl.pallas_call(
        _mlp_body,
        grid=grid,
        in_specs=[
            pl.BlockSpec((B, _TN, _P), lambda i: (0, i, 0)),
            nspec, nspec, nspec,
            pl.BlockSpec((B, _TN, KC), lambda i: (0, i, 0)),
            gspec, gspec, gspec, gspec,
            pl.BlockSpec((_P, HID), lambda i: (0, 0)),
            pl.BlockSpec((K, HID), lambda i: (0, 0)),
            pl.BlockSpec((K, HID), lambda i: (0, 0)),
            pl.BlockSpec((K, HID), lambda i: (0, 0)),
            pl.BlockSpec((1, HID), lambda i: (0, 0)),
            pl.BlockSpec((HID, OUTD), lambda i: (0, 0)),
            pl.BlockSpec((1, OUTD), lambda i: (0, 0)),
        ],
        out_specs=pl.BlockSpec((B, _TN, C), lambda i: (0, i, 0)),
        out_shape=jax.ShapeDtypeStruct((B, N, C), jnp.float32),
    )(pts_p, nbr0, nbr1, nbr2, feats, g1, be1, g2, be2,
      w1c, w10, w11, w12, b1, w2t, b2)


def kernel(x, points, indices, W1, b1, g1, be1, W2, b2, g2, be2):
    B, N, C = x.shape
    K = indices.shape[2]
    HID = W1.shape[0]
    OUTD = W2.shape[0]

    xt = x.reshape(B * N, C)
    pf = points.reshape(B * N, 3)
    idx = (indices.astype(jnp.int32)
           + (jnp.arange(B, dtype=jnp.int32) * N)[:, None, None])
    idx = idx.reshape(B * N * K)
    feats = _sc_gather_feats(xt, idx.reshape(1, -1)).reshape(B, N, K * C)
    n0, n1, n2 = _sc_gather_coords(pf[:, 0], pf[:, 1], pf[:, 2], idx)
    n0 = n0.reshape(B, N, K)
    n1 = n1.reshape(B, N, K)
    n2 = n2.reshape(B, N, K)

    pts_p = jnp.pad(points, ((0, 0), (0, 0), (0, _P - 3)))
    w1_khj = W1.reshape(HID, K, 3)
    w10 = w1_khj[:, :, 0].T
    w11 = w1_khj[:, :, 1].T
    w12 = w1_khj[:, :, 2].T
    w1c = jnp.zeros((_P, HID), W1.dtype).at[:3, :].set(
        jnp.sum(w1_khj, axis=1).T)
    w2t = W2.T

    out = _mlp_call(pts_p, n0, n1, n2, feats,
                    g1.reshape(N, 1), be1.reshape(N, 1),
                    g2.reshape(N, 1), be2.reshape(N, 1),
                    w1c, w10, w11, w12, b1.reshape(1, HID), w2t,
                    b2.reshape(1, OUTD))
    return (out, points, indices)


# speedup vs baseline: 18.4899x; 1.0079x over previous
import dataclasses
import functools

import jax
import jax.numpy as jnp
from jax.experimental import pallas as pl
from jax.experimental.pallas import tpu as pltpu
from jax.experimental.pallas import tpu_sc as plsc

_P = 8
_TN = 200
_GW = 128
_NW = 32
_VEC = 16


def _sc_gather_feats(xt, idx):
    M = idx.shape[1]
    C = xt.shape[1]
    mesh = plsc.VectorSubcoreMesh(core_axis_name="c", subcore_axis_name="s")

    @functools.partial(
        pl.kernel,
        out_type=jax.ShapeDtypeStruct((M, C), xt.dtype),
        mesh=mesh,
    )
    def k(x_hbm, i_hbm, o_hbm):
        def body(i_vmem, o_vmem):
            pltpu.sync_copy(x_hbm.at[i_vmem.at[0]], o_vmem)

        pltpu.emit_pipeline(
            body,
            grid=(M // _GW,),
            in_specs=[pl.BlockSpec((1, _GW), lambda i: (0, i))],
            out_specs=[pl.BlockSpec((_GW, C), lambda i: (i, 0))],
            core_axis_name=("c", "s"),
            dimension_semantics=(pltpu.PARALLEL,),
        )(i_hbm, o_hbm)

    return k(xt, idx)


def _sc_gather_coords(px, py, pz, idx):
    R = px.shape[0]
    M = idx.shape[0]
    per_w = M // _NW
    CH = 2000
    mesh = plsc.VectorSubcoreMesh(core_axis_name="c", subcore_axis_name="s")
    cp = pltpu.CompilerParams()
    if "needs_layout_passes" in pltpu.CompilerParams.__dataclass_fields__:
        cp = dataclasses.replace(cp, needs_layout_passes=False)

    @functools.partial(
        pl.kernel,
        out_type=tuple(jax.ShapeDtypeStruct((M,), jnp.float32)
                       for _ in range(3)),
        mesh=mesh,
        compiler_params=cp,
        scratch_types=[
            pltpu.VMEM((R,), jnp.float32),
            pltpu.VMEM((R,), jnp.float32),
            pltpu.VMEM((R,), jnp.float32),
            pltpu.VMEM((CH,), jnp.int32),
            pltpu.VMEM((CH,), jnp.float32),
            pltpu.VMEM((CH,), jnp.float32),
            pltpu.VMEM((CH,), jnp.float32),
        ],
    )
    def k(px_hbm, py_hbm, pz_hbm, i_hbm, o0_hbm, o1_hbm, o2_hbm,
          tx_v, ty_v, tz_v, i_v, o0_v, o1_v, o2_v):
        wid = jax.lax.axis_index("s") * 2 + jax.lax.axis_index("c")
        base = wid * per_w
        pltpu.sync_copy(px_hbm, tx_v)
        pltpu.sync_copy(py_hbm, ty_v)
        pltpu.sync_copy(pz_hbm, tz_v)

        @pl.loop(0, per_w, step=CH)
        def _chunk(c0):
            pltpu.sync_copy(i_hbm.at[pl.ds(base + c0, CH)], i_v)

            @pl.loop(0, CH, step=_VEC)
            def _vec(t):
                iv = i_v[pl.ds(t, _VEC)]
                o0_v[pl.ds(t, _VEC)] = plsc.load_gather(tx_v, [iv])
                o1_v[pl.ds(t, _VEC)] = plsc.load_gather(ty_v, [iv])
                o2_v[pl.ds(t, _VEC)] = plsc.load_gather(tz_v, [iv])

            pltpu.sync_copy(o0_v, o0_hbm.at[pl.ds(base + c0, CH)])
            pltpu.sync_copy(o1_v, o1_hbm.at[pl.ds(base + c0, CH)])
            pltpu.sync_copy(o2_v, o2_hbm.at[pl.ds(base + c0, CH)])

    return k(px, py, pz, idx)


def _bn_relu(y, g_ref, be_ref):
    cnt = y.shape[0] * y.shape[2]
    s = jnp.sum(y, axis=2, keepdims=True)
    ss = jnp.sum(y * y, axis=2, keepdims=True)
    mean = jnp.sum(s, axis=0, keepdims=True) / cnt
    ex2 = jnp.sum(ss, axis=0, keepdims=True) / cnt
    var = ex2 - mean * mean
    inv = jax.lax.rsqrt(var + 1e-5)
    g = g_ref[...][None]
    be = be_ref[...][None]
    return jnp.maximum((y - mean) * (inv * g) + be, 0.0)


def _mlp_body(pts_ref, n0_ref, n1_ref, n2_ref, feat_ref,
              g1_ref, be1_ref, g2_ref, be2_ref,
              w1c_ref, w10_ref, w11_ref, w12_ref, b1_ref, w2_ref, b2_ref,
              out_ref):
    B, TN, _ = pts_ref.shape
    HID = w1c_ref.shape[1]
    OUTD = w2_ref.shape[1]
    C = out_ref.shape[2]
    K = OUTD // C

    p = pts_ref[...].reshape(B * TN, _P)
    dot = functools.partial(jnp.dot, preferred_element_type=jnp.float32)
    y = (dot(p, w1c_ref[...])
         - dot(n0_ref[...].reshape(B * TN, K), w10_ref[...])
         - dot(n1_ref[...].reshape(B * TN, K), w11_ref[...])
         - dot(n2_ref[...].reshape(B * TN, K), w12_ref[...])
         + b1_ref[...])
    y = _bn_relu(y.reshape(B, TN, HID), g1_ref, be1_ref)
    z = (dot(y.reshape(B * TN, HID).astype(w2_ref.dtype), w2_ref[...])
         + b2_ref[...])
    z = _bn_relu(z.reshape(B, TN, OUTD), g2_ref, be2_ref)

    f = feat_ref[...]
    acc = z[:, :, 0:C] * f[:, :, 0:C]
    for k in range(1, K):
        acc = acc + z[:, :, k * C:(k + 1) * C] * f[:, :, k * C:(k + 1) * C]
    out_ref[...] = acc


def _mlp_call(pts_p, nbr0, nbr1, nbr2, feats, g1, be1, g2, be2,
              w1c, w10, w11, w12, b1, w2t, b2):
    B, N, _ = pts_p.shape
    K = nbr0.shape[2]
    KC = feats.shape[2]
    HID = w2t.shape[0]
    OUTD = w2t.shape[1]
    C = KC // K
    grid = (N // _TN,)
    nspec = pl.BlockSpec((B, _TN, K), lambda i: (0, i, 0))
    gspec = pl.BlockSpec((_TN, 1), lambda i: (i, 0))
    return pl.pallas_call(
        _mlp_body,
        grid=grid,
        in_specs=[
            pl.BlockSpec((B, _TN, _P), lambda i: (0, i, 0)),
            nspec, nspec, nspec,
            pl.BlockSpec((B, _TN, KC), lambda i: (0, i, 0)),
            gspec, gspec, gspec, gspec,
            pl.BlockSpec((_P, HID), lambda i: (0, 0)),
            pl.BlockSpec((K, HID), lambda i: (0, 0)),
            pl.BlockSpec((K, HID), lambda i: (0, 0)),
            pl.BlockSpec((K, HID), lambda i: (0, 0)),
            pl.BlockSpec((1, HID), lambda i: (0, 0)),
            pl.BlockSpec((HID, OUTD), lambda i: (0, 0)),
            pl.BlockSpec((1, OUTD), lambda i: (0, 0)),
        ],
        out_specs=pl.BlockSpec((B, _TN, C), lambda i: (0, i, 0)),
        out_shape=jax.ShapeDtypeStruct((B, N, C), jnp.float32),
    )(pts_p, nbr0, nbr1, nbr2, feats, g1, be1, g2, be2,
      w1c, w10, w11, w12, b1, w2t, b2)


def kernel(x, points, indices, W1, b1, g1, be1, W2, b2, g2, be2):
    B, N, C = x.shape
    K = indices.shape[2]
    HID = W1.shape[0]
    OUTD = W2.shape[0]

    xt = x.reshape(B * N, C)
    pf = points.reshape(B * N, 3)
    idx = (indices.astype(jnp.int32)
           + (jnp.arange(B, dtype=jnp.int32) * N)[:, None, None])
    idx = idx.reshape(B * N * K)
    feats = _sc_gather_feats(xt, idx.reshape(1, -1)).reshape(B, N, K * C)
    n0, n1, n2 = _sc_gather_coords(pf[:, 0], pf[:, 1], pf[:, 2], idx)
    n0 = n0.reshape(B, N, K)
    n1 = n1.reshape(B, N, K)
    n2 = n2.reshape(B, N, K)

    pts_p = jnp.pad(points, ((0, 0), (0, 0), (0, _P - 3)))
    w1_khj = W1.reshape(HID, K, 3)
    w10 = w1_khj[:, :, 0].T
    w11 = w1_khj[:, :, 1].T
    w12 = w1_khj[:, :, 2].T
    w1c = jnp.zeros((_P, HID), W1.dtype).at[:3, :].set(
        jnp.sum(w1_khj, axis=1).T)
    w2t = W2.T.astype(jnp.bfloat16)

    out = _mlp_call(pts_p, n0, n1, n2, feats,
                    g1.reshape(N, 1), be1.reshape(N, 1),
                    g2.reshape(N, 1), be2.reshape(N, 1),
                    w1c, w10, w11, w12, b1.reshape(1, HID), w2t,
                    b2.reshape(1, OUTD))
    return (out, points, indices)


# speedup vs baseline: 22.0040x; 1.1901x over previous
import dataclasses
import functools

import jax
import jax.numpy as jnp
from jax.experimental import pallas as pl
from jax.experimental.pallas import tpu as pltpu
from jax.experimental.pallas import tpu_sc as plsc

_P = 8
_TN = 200
_GW = 128
_NW = 32
_VEC = 16


def _sc_gather_feats(xt, idx):
    M = idx.shape[1]
    C = xt.shape[1]
    mesh = plsc.VectorSubcoreMesh(core_axis_name="c", subcore_axis_name="s")

    @functools.partial(
        pl.kernel,
        out_type=jax.ShapeDtypeStruct((M, C), xt.dtype),
        mesh=mesh,
    )
    def k(x_hbm, i_hbm, o_hbm):
        def body(i_vmem, o_vmem):
            pltpu.sync_copy(x_hbm.at[i_vmem.at[0]], o_vmem)

        pltpu.emit_pipeline(
            body,
            grid=(M // _GW,),
            in_specs=[pl.BlockSpec((1, _GW), lambda i: (0, i))],
            out_specs=[pl.BlockSpec((_GW, C), lambda i: (i, 0))],
            core_axis_name=("c", "s"),
            dimension_semantics=(pltpu.PARALLEL,),
        )(i_hbm, o_hbm)

    return k(xt, idx)


def _sc_gather_coords(px, py, pz, idx):
    R = px.shape[0]
    M = idx.shape[0]
    per_w = M // _NW
    CH = 2000
    mesh = plsc.VectorSubcoreMesh(core_axis_name="c", subcore_axis_name="s")
    cp = pltpu.CompilerParams()
    if "needs_layout_passes" in pltpu.CompilerParams.__dataclass_fields__:
        cp = dataclasses.replace(cp, needs_layout_passes=False)

    @functools.partial(
        pl.kernel,
        out_type=tuple(jax.ShapeDtypeStruct((M,), jnp.float32)
                       for _ in range(3)),
        mesh=mesh,
        compiler_params=cp,
        scratch_types=[
            pltpu.VMEM((R,), jnp.float32),
            pltpu.VMEM((R,), jnp.float32),
            pltpu.VMEM((R,), jnp.float32),
            pltpu.VMEM((CH,), jnp.int32),
            pltpu.VMEM((CH,), jnp.float32),
            pltpu.VMEM((CH,), jnp.float32),
            pltpu.VMEM((CH,), jnp.float32),
        ],
    )
    def k(px_hbm, py_hbm, pz_hbm, i_hbm, o0_hbm, o1_hbm, o2_hbm,
          tx_v, ty_v, tz_v, i_v, o0_v, o1_v, o2_v):
        wid = jax.lax.axis_index("s") * 2 + jax.lax.axis_index("c")
        base = wid * per_w
        pltpu.sync_copy(px_hbm, tx_v)
        pltpu.sync_copy(py_hbm, ty_v)
        pltpu.sync_copy(pz_hbm, tz_v)

        @pl.loop(0, per_w, step=CH)
        def _chunk(c0):
            pltpu.sync_copy(i_hbm.at[pl.ds(base + c0, CH)], i_v)

            @pl.loop(0, CH, step=_VEC)
            def _vec(t):
                iv = i_v[pl.ds(t, _VEC)]
                o0_v[pl.ds(t, _VEC)] = plsc.load_gather(tx_v, [iv])
                o1_v[pl.ds(t, _VEC)] = plsc.load_gather(ty_v, [iv])
                o2_v[pl.ds(t, _VEC)] = plsc.load_gather(tz_v, [iv])

            pltpu.sync_copy(o0_v, o0_hbm.at[pl.ds(base + c0, CH)])
            pltpu.sync_copy(o1_v, o1_hbm.at[pl.ds(base + c0, CH)])
            pltpu.sync_copy(o2_v, o2_hbm.at[pl.ds(base + c0, CH)])

    return k(px, py, pz, idx)


def _bn_relu(y, g_ref, be_ref):
    cnt = y.shape[0] * y.shape[2]
    s = jnp.sum(y, axis=2, keepdims=True)
    ss = jnp.sum(y * y, axis=2, keepdims=True)
    mean = jnp.sum(s, axis=0, keepdims=True) / cnt
    ex2 = jnp.sum(ss, axis=0, keepdims=True) / cnt
    var = ex2 - mean * mean
    inv = jax.lax.rsqrt(var + 1e-5)
    g = g_ref[...][None]
    be = be_ref[...][None]
    return jnp.maximum((y - mean) * (inv * g) + be, 0.0)


def _mlp_body(pts_ref, n0_ref, n1_ref, n2_ref, feat_ref, bn_ref,
              w1c_ref, w10_ref, w11_ref, w12_ref, b1_ref, w2_ref, b2_ref,
              out_ref):
    B, TN, _ = pts_ref.shape
    HID = w1c_ref.shape[1]
    OUTD = w2_ref.shape[0]
    C = out_ref.shape[2]
    K = OUTD // C

    p = pts_ref[...].reshape(B * TN, _P)
    dot = functools.partial(jnp.dot, preferred_element_type=jnp.float32)
    y = (dot(p, w1c_ref[...])
         - dot(n0_ref[...].reshape(B * TN, K), w10_ref[...])
         - dot(n1_ref[...].reshape(B * TN, K), w11_ref[...])
         - dot(n2_ref[...].reshape(B * TN, K), w12_ref[...])
         + b1_ref[...])
    y = _bn_relu(y.reshape(B, TN, HID), bn_ref[:, 0:1], bn_ref[:, 1:2])
    z = jax.lax.dot_general(
        y.reshape(B * TN, HID).astype(w2_ref.dtype), w2_ref[...],
        (((1,), (1,)), ((), ())),
        preferred_element_type=jnp.float32) + b2_ref[...]
    z = _bn_relu(z.reshape(B, TN, OUTD), bn_ref[:, 2:3], bn_ref[:, 3:4])

    f = feat_ref[...]
    acc = z[:, :, 0:C] * f[:, :, 0, :]
    for k in range(1, K):
        acc = acc + z[:, :, k * C:(k + 1) * C] * f[:, :, k, :]
    out_ref[...] = acc


def _mlp_call(pts_p, nbr0, nbr1, nbr2, feats, bn, w1c, w10, w11, w12,
              b1, w2, b2):
    B, N, _ = pts_p.shape
    K = nbr0.shape[2]
    C = feats.shape[3]
    HID = w2.shape[1]
    OUTD = w2.shape[0]
    grid = (N // _TN,)
    nspec = pl.BlockSpec((B, _TN, K), lambda i: (0, i, 0))
    return pl.pallas_call(
        _mlp_body,
        grid=grid,
        in_specs=[
            pl.BlockSpec((B, _TN, _P), lambda i: (0, i, 0)),
            nspec, nspec, nspec,
            pl.BlockSpec((B, _TN, K, C), lambda i: (0, i, 0, 0)),
            pl.BlockSpec((_TN, 4), lambda i: (i, 0)),
            pl.BlockSpec((_P, HID), lambda i: (0, 0)),
            pl.BlockSpec((K, HID), lambda i: (0, 0)),
            pl.BlockSpec((K, HID), lambda i: (0, 0)),
            pl.BlockSpec((K, HID), lambda i: (0, 0)),
            pl.BlockSpec((1, HID), lambda i: (0, 0)),
            pl.BlockSpec((OUTD, HID), lambda i: (0, 0)),
            pl.BlockSpec((1, OUTD), lambda i: (0, 0)),
        ],
        out_specs=pl.BlockSpec((B, _TN, C), lambda i: (0, i, 0)),
        out_shape=jax.ShapeDtypeStruct((B, N, C), jnp.float32),
    )(pts_p, nbr0, nbr1, nbr2, feats, bn, w1c, w10, w11, w12, b1, w2, b2)


def kernel(x, points, indices, W1, b1, g1, be1, W2, b2, g2, be2):
    B, N, C = x.shape
    K = indices.shape[2]
    HID = W1.shape[0]
    OUTD = W2.shape[0]

    xt = x.reshape(B * N, C)
    pf = points.reshape(B * N, 3)
    idx = (indices.astype(jnp.int32)
           + (jnp.arange(B, dtype=jnp.int32) * N)[:, None, None])
    idx = idx.reshape(B * N * K)
    feats = _sc_gather_feats(xt, idx.reshape(1, -1)).reshape(B, N, K, C)
    n0, n1, n2 = _sc_gather_coords(pf[:, 0], pf[:, 1], pf[:, 2], idx)
    n0 = n0.reshape(B, N, K)
    n1 = n1.reshape(B, N, K)
    n2 = n2.reshape(B, N, K)

    pts_p = jnp.pad(points, ((0, 0), (0, 0), (0, _P - 3)))
    w1_khj = W1.reshape(HID, K, 3)
    w10 = w1_khj[:, :, 0].T
    w11 = w1_khj[:, :, 1].T
    w12 = w1_khj[:, :, 2].T
    w1c = jnp.zeros((_P, HID), W1.dtype).at[:3, :].set(
        jnp.sum(w1_khj, axis=1).T)
    w2b = W2.astype(jnp.bfloat16)
    bn = jnp.stack([g1, be1, g2, be2], axis=-1)

    out = _mlp_call(pts_p, n0, n1, n2, feats, bn,
                    w1c, w10, w11, w12, b1.reshape(1, HID), w2b,
                    b2.reshape(1, OUTD))
    return (out, points, indices)


# speedup vs baseline: 29.4961x; 1.3405x over previous
import dataclasses
import functools

import jax
import jax.numpy as jnp
from jax.experimental import pallas as pl
from jax.experimental.pallas import tpu as pltpu
from jax.experimental.pallas import tpu_sc as plsc

_P = 8
_TN = 200
_GW = 128
_NW = 32
_VEC = 16
_NCHUNK = 2


def _sc_gather_feats(xt, idx):
    M = idx.shape[1]
    C = xt.shape[1]
    mesh = plsc.VectorSubcoreMesh(core_axis_name="c", subcore_axis_name="s")

    @functools.partial(
        pl.kernel,
        out_type=jax.ShapeDtypeStruct((M, C), xt.dtype),
        mesh=mesh,
    )
    def k(x_hbm, i_hbm, o_hbm):
        def body(i_vmem, o_vmem):
            pltpu.sync_copy(x_hbm.at[i_vmem.at[0]], o_vmem)

        pltpu.emit_pipeline(
            body,
            grid=(M // _GW,),
            in_specs=[pl.BlockSpec((1, _GW), lambda i: (0, i))],
            out_specs=[pl.BlockSpec((_GW, C), lambda i: (i, 0))],
            core_axis_name=("c", "s"),
            dimension_semantics=(pltpu.PARALLEL,),
        )(i_hbm, o_hbm)

    return k(xt, idx)


def _sc_gather_coords(px, py, pz, idx):
    R = px.shape[0]
    M = idx.shape[0]
    per_w = M // _NW
    CH = 2000
    mesh = plsc.VectorSubcoreMesh(core_axis_name="c", subcore_axis_name="s")
    cp = pltpu.CompilerParams()
    if "needs_layout_passes" in pltpu.CompilerParams.__dataclass_fields__:
        cp = dataclasses.replace(cp, needs_layout_passes=False)

    @functools.partial(
        pl.kernel,
        out_type=tuple(jax.ShapeDtypeStruct((M,), jnp.float32)
                       for _ in range(3)),
        mesh=mesh,
        compiler_params=cp,
        scratch_types=[
            pltpu.VMEM((R,), jnp.float32),
            pltpu.VMEM((R,), jnp.float32),
            pltpu.VMEM((R,), jnp.float32),
            pltpu.VMEM((CH,), jnp.int32),
            pltpu.VMEM((CH,), jnp.float32),
            pltpu.VMEM((CH,), jnp.float32),
            pltpu.VMEM((CH,), jnp.float32),
        ],
    )
    def k(px_hbm, py_hbm, pz_hbm, i_hbm, o0_hbm, o1_hbm, o2_hbm,
          tx_v, ty_v, tz_v, i_v, o0_v, o1_v, o2_v):
        wid = jax.lax.axis_index("s") * 2 + jax.lax.axis_index("c")
        base = wid * per_w
        pltpu.sync_copy(px_hbm, tx_v)
        pltpu.sync_copy(py_hbm, ty_v)
        pltpu.sync_copy(pz_hbm, tz_v)

        @pl.loop(0, per_w, step=CH)
        def _chunk(c0):
            pltpu.sync_copy(i_hbm.at[pl.ds(base + c0, CH)], i_v)

            @pl.loop(0, CH, step=_VEC)
            def _vec(t):
                iv = i_v[pl.ds(t, _VEC)]
                o0_v[pl.ds(t, _VEC)] = plsc.load_gather(tx_v, [iv])
                o1_v[pl.ds(t, _VEC)] = plsc.load_gather(ty_v, [iv])
                o2_v[pl.ds(t, _VEC)] = plsc.load_gather(tz_v, [iv])

            pltpu.sync_copy(o0_v, o0_hbm.at[pl.ds(base + c0, CH)])
            pltpu.sync_copy(o1_v, o1_hbm.at[pl.ds(base + c0, CH)])
            pltpu.sync_copy(o2_v, o2_hbm.at[pl.ds(base + c0, CH)])

    return k(px, py, pz, idx)


def _bn_relu(y, g, be):
    cnt = y.shape[0] * y.shape[2]
    s = jnp.sum(y, axis=2, keepdims=True)
    ss = jnp.sum(y * y, axis=2, keepdims=True)
    mean = jnp.sum(s, axis=0, keepdims=True) / cnt
    ex2 = jnp.sum(ss, axis=0, keepdims=True) / cnt
    var = ex2 - mean * mean
    inv = jax.lax.rsqrt(var + 1e-5)
    return jnp.maximum((y - mean) * (inv * g[None]) + be[None], 0.0)


def _mlp_body(pts_ref, n0_ref, n1_ref, n2_ref, feat_ref, bn_ref,
              w1c_ref, w10_ref, w11_ref, w12_ref, b1_ref, w2_ref, b2_ref,
              out_ref):
    B, TN, _ = pts_ref.shape
    HID = w1c_ref.shape[1]
    OUTD = w2_ref.shape[1]
    C = out_ref.shape[2]
    K = OUTD // C

    p = pts_ref[...].reshape(B * TN, _P)
    dot = functools.partial(jnp.dot, preferred_element_type=jnp.float32)
    y = (dot(p, w1c_ref[...])
         - dot(n0_ref[...].reshape(B * TN, K), w10_ref[...])
         - dot(n1_ref[...].reshape(B * TN, K), w11_ref[...])
         - dot(n2_ref[...].reshape(B * TN, K), w12_ref[...])
         + b1_ref[...])
    y = _bn_relu(y.reshape(B, TN, HID), bn_ref[:, 0:1], bn_ref[:, 1:2])
    z = (dot(y.reshape(B * TN, HID).astype(w2_ref.dtype), w2_ref[...])
         + b2_ref[...])
    z = _bn_relu(z.reshape(B, TN, OUTD), bn_ref[:, 2:3], bn_ref[:, 3:4])

    f = feat_ref[...]
    acc = z[:, :, 0:C] * f[:, 0]
    for k in range(1, K):
        acc = acc + z[:, :, k * C:(k + 1) * C] * f[:, k]
    out_ref[...] = acc


def _mlp_call(pts_p, nbr0, nbr1, nbr2, feats, bn, w1c, w10, w11, w12,
              b1, w2t, b2, tile0, ntiles):
    B = pts_p.shape[0]
    K = nbr0.shape[2]
    C = feats.shape[3]
    HID = w2t.shape[0]
    OUTD = w2t.shape[1]
    nspec = pl.BlockSpec((B, _TN, K), lambda i: (0, i + tile0, 0))
    return pl.pallas_call(
        _mlp_body,
        grid=(ntiles,),
        in_specs=[
            pl.BlockSpec((B, _TN, _P), lambda i: (0, i + tile0, 0)),
            nspec, nspec, nspec,
            pl.BlockSpec((B, K, _TN, C), lambda i: (0, 0, i, 0)),
            pl.BlockSpec((_TN, 4), lambda i: (i + tile0, 0)),
            pl.BlockSpec((_P, HID), lambda i: (0, 0)),
            pl.BlockSpec((K, HID), lambda i: (0, 0)),
            pl.BlockSpec((K, HID), lambda i: (0, 0)),
            pl.BlockSpec((K, HID), lambda i: (0, 0)),
            pl.BlockSpec((1, HID), lambda i: (0, 0)),
            pl.BlockSpec((HID, OUTD), lambda i: (0, 0)),
            pl.BlockSpec((1, OUTD), lambda i: (0, 0)),
        ],
        out_specs=pl.BlockSpec((B, _TN, C), lambda i: (0, i, 0)),
        out_shape=jax.ShapeDtypeStruct((B, ntiles * _TN, C), jnp.float32),
    )(pts_p, nbr0, nbr1, nbr2, feats, bn, w1c, w10, w11, w12, b1, w2t, b2)


def kernel(x, points, indices, W1, b1, g1, be1, W2, b2, g2, be2):
    B, N, C = x.shape
    K = indices.shape[2]
    HID = W1.shape[0]
    OUTD = W2.shape[0]

    xt = x.reshape(B * N, C)
    pf = points.reshape(B * N, 3)
    idx = (indices.astype(jnp.int32)
           + (jnp.arange(B, dtype=jnp.int32) * N)[:, None, None])
    n0, n1, n2 = _sc_gather_coords(pf[:, 0], pf[:, 1], pf[:, 2],
                                   idx.reshape(B * N * K))
    n0 = n0.reshape(B, N, K)
    n1 = n1.reshape(B, N, K)
    n2 = n2.reshape(B, N, K)

    pts_p = jnp.pad(points, ((0, 0), (0, 0), (0, _P - 3)))
    w1_khj = W1.reshape(HID, K, 3)
    w10 = w1_khj[:, :, 0].T
    w11 = w1_khj[:, :, 1].T
    w12 = w1_khj[:, :, 2].T
    w1c = jnp.zeros((_P, HID), W1.dtype).at[:3, :].set(
        jnp.sum(w1_khj, axis=1).T)
    w2t = W2.T.astype(jnp.bfloat16)
    bn = jnp.stack([g1, be1, g2, be2], axis=-1)

    nc = N // _NCHUNK
    tiles_per_chunk = nc // _TN
    outs = []
    for c in range(_NCHUNK):
        idx_c = idx[:, c * nc:(c + 1) * nc, :].transpose(0, 2, 1)
        feats_c = _sc_gather_feats(
            xt, idx_c.reshape(1, B * K * nc)).reshape(B, K, nc, C)
        outs.append(_mlp_call(
            pts_p, n0, n1, n2, feats_c, bn, w1c, w10, w11, w12,
            b1.reshape(1, HID), w2t, b2.reshape(1, OUTD),
            c * tiles_per_chunk, tiles_per_chunk))
    out = jnp.concatenate(outs, axis=1)
    return (out, points, indices)


# speedup vs baseline: 30.3690x; 1.0296x over previous
import dataclasses
import functools

import jax
import jax.numpy as jnp
from jax.experimental import pallas as pl
from jax.experimental.pallas import tpu as pltpu
from jax.experimental.pallas import tpu_sc as plsc

_P = 8
_TN = 200
_GW = 128
_NW = 32
_VEC = 16
_NCHUNK = 5


def _sc_gather_feats(xt, idx):
    M = idx.shape[1]
    C = xt.shape[1]
    mesh = plsc.VectorSubcoreMesh(core_axis_name="c", subcore_axis_name="s")

    @functools.partial(
        pl.kernel,
        out_type=jax.ShapeDtypeStruct((M, C), xt.dtype),
        mesh=mesh,
    )
    def k(x_hbm, i_hbm, o_hbm):
        def body(i_vmem, o_vmem):
            pltpu.sync_copy(x_hbm.at[i_vmem.at[0]], o_vmem)

        pltpu.emit_pipeline(
            body,
            grid=(M // _GW,),
            in_specs=[pl.BlockSpec((1, _GW), lambda i: (0, i))],
            out_specs=[pl.BlockSpec((_GW, C), lambda i: (i, 0))],
            core_axis_name=("c", "s"),
            dimension_semantics=(pltpu.PARALLEL,),
        )(i_hbm, o_hbm)

    return k(xt, idx)


def _sc_gather_coords(px, py, pz, idx):
    R = px.shape[0]
    M = idx.shape[0]
    per_w = M // _NW
    CH = 2000
    mesh = plsc.VectorSubcoreMesh(core_axis_name="c", subcore_axis_name="s")
    cp = pltpu.CompilerParams()
    if "needs_layout_passes" in pltpu.CompilerParams.__dataclass_fields__:
        cp = dataclasses.replace(cp, needs_layout_passes=False)

    @functools.partial(
        pl.kernel,
        out_type=tuple(jax.ShapeDtypeStruct((M,), jnp.float32)
                       for _ in range(3)),
        mesh=mesh,
        compiler_params=cp,
        scratch_types=[
            pltpu.VMEM((R,), jnp.float32),
            pltpu.VMEM((R,), jnp.float32),
            pltpu.VMEM((R,), jnp.float32),
            pltpu.VMEM((CH,), jnp.int32),
            pltpu.VMEM((CH,), jnp.float32),
            pltpu.VMEM((CH,), jnp.float32),
            pltpu.VMEM((CH,), jnp.float32),
        ],
    )
    def k(px_hbm, py_hbm, pz_hbm, i_hbm, o0_hbm, o1_hbm, o2_hbm,
          tx_v, ty_v, tz_v, i_v, o0_v, o1_v, o2_v):
        wid = jax.lax.axis_index("s") * 2 + jax.lax.axis_index("c")
        base = wid * per_w
        pltpu.sync_copy(px_hbm, tx_v)
        pltpu.sync_copy(py_hbm, ty_v)
        pltpu.sync_copy(pz_hbm, tz_v)

        @pl.loop(0, per_w, step=CH)
        def _chunk(c0):
            pltpu.sync_copy(i_hbm.at[pl.ds(base + c0, CH)], i_v)

            @pl.loop(0, CH, step=_VEC)
            def _vec(t):
                iv = i_v[pl.ds(t, _VEC)]
                o0_v[pl.ds(t, _VEC)] = plsc.load_gather(tx_v, [iv])
                o1_v[pl.ds(t, _VEC)] = plsc.load_gather(ty_v, [iv])
                o2_v[pl.ds(t, _VEC)] = plsc.load_gather(tz_v, [iv])

            pltpu.sync_copy(o0_v, o0_hbm.at[pl.ds(base + c0, CH)])
            pltpu.sync_copy(o1_v, o1_hbm.at[pl.ds(base + c0, CH)])
            pltpu.sync_copy(o2_v, o2_hbm.at[pl.ds(base + c0, CH)])

    return k(px, py, pz, idx)


def _bn_relu(y, g, be):
    cnt = y.shape[0] * y.shape[2]
    s = jnp.sum(y, axis=2, keepdims=True)
    ss = jnp.sum(y * y, axis=2, keepdims=True)
    mean = jnp.sum(s, axis=0, keepdims=True) / cnt
    ex2 = jnp.sum(ss, axis=0, keepdims=True) / cnt
    var = ex2 - mean * mean
    inv = jax.lax.rsqrt(var + 1e-5)
    return jnp.maximum((y - mean) * (inv * g[None]) + be[None], 0.0)


def _mlp_body(pts_ref, n0_ref, n1_ref, n2_ref, feat_ref, bn_ref,
              w1c_ref, w10_ref, w11_ref, w12_ref, b1_ref, w2_ref, b2_ref,
              out_ref):
    B, TN, _ = pts_ref.shape
    HID = w1c_ref.shape[1]
    OUTD = w2_ref.shape[1]
    C = out_ref.shape[2]
    K = OUTD // C

    p = pts_ref[...].reshape(B * TN, _P)
    dot = functools.partial(jnp.dot, preferred_element_type=jnp.float32)
    y = (dot(p, w1c_ref[...])
         - dot(n0_ref[...].reshape(B * TN, K), w10_ref[...])
         - dot(n1_ref[...].reshape(B * TN, K), w11_ref[...])
         - dot(n2_ref[...].reshape(B * TN, K), w12_ref[...])
         + b1_ref[...])
    y = _bn_relu(y.reshape(B, TN, HID), bn_ref[:, 0:1], bn_ref[:, 1:2])
    z = (dot(y.reshape(B * TN, HID).astype(w2_ref.dtype), w2_ref[...])
         + b2_ref[...])
    z = _bn_relu(z.reshape(B, TN, OUTD), bn_ref[:, 2:3], bn_ref[:, 3:4])

    f = feat_ref[...]
    acc = z[:, :, 0:C] * f[:, 0]
    for k in range(1, K):
        acc = acc + z[:, :, k * C:(k + 1) * C] * f[:, k]
    out_ref[...] = acc


def _mlp_call(pts_p, nbr0, nbr1, nbr2, feats, bn, w1c, w10, w11, w12,
              b1, w2t, b2, tile0, ntiles):
    B = pts_p.shape[0]
    K = nbr0.shape[2]
    C = feats.shape[3]
    HID = w2t.shape[0]
    OUTD = w2t.shape[1]
    nspec = pl.BlockSpec((B, _TN, K), lambda i: (0, i + tile0, 0))
    return pl.pallas_call(
        _mlp_body,
        grid=(ntiles,),
        in_specs=[
            pl.BlockSpec((B, _TN, _P), lambda i: (0, i + tile0, 0)),
            nspec, nspec, nspec,
            pl.BlockSpec((B, K, _TN, C), lambda i: (0, 0, i, 0)),
            pl.BlockSpec((_TN, 4), lambda i: (i + tile0, 0)),
            pl.BlockSpec((_P, HID), lambda i: (0, 0)),
            pl.BlockSpec((K, HID), lambda i: (0, 0)),
            pl.BlockSpec((K, HID), lambda i: (0, 0)),
            pl.BlockSpec((K, HID), lambda i: (0, 0)),
            pl.BlockSpec((1, HID), lambda i: (0, 0)),
            pl.BlockSpec((HID, OUTD), lambda i: (0, 0)),
            pl.BlockSpec((1, OUTD), lambda i: (0, 0)),
        ],
        out_specs=pl.BlockSpec((B, _TN, C), lambda i: (0, i, 0)),
        out_shape=jax.ShapeDtypeStruct((B, ntiles * _TN, C), jnp.float32),
    )(pts_p, nbr0, nbr1, nbr2, feats, bn, w1c, w10, w11, w12, b1, w2t, b2)


def kernel(x, points, indices, W1, b1, g1, be1, W2, b2, g2, be2):
    B, N, C = x.shape
    K = indices.shape[2]
    HID = W1.shape[0]
    OUTD = W2.shape[0]

    xt = x.reshape(B * N, C)
    pf = points.reshape(B * N, 3)
    idx = (indices.astype(jnp.int32)
           + (jnp.arange(B, dtype=jnp.int32) * N)[:, None, None])
    n0, n1, n2 = _sc_gather_coords(pf[:, 0], pf[:, 1], pf[:, 2],
                                   idx.reshape(B * N * K))
    n0 = n0.reshape(B, N, K)
    n1 = n1.reshape(B, N, K)
    n2 = n2.reshape(B, N, K)

    pts_p = jnp.pad(points, ((0, 0), (0, 0), (0, _P - 3)))
    w1_khj = W1.reshape(HID, K, 3)
    w10 = w1_khj[:, :, 0].T
    w11 = w1_khj[:, :, 1].T
    w12 = w1_khj[:, :, 2].T
    w1c = jnp.zeros((_P, HID), W1.dtype).at[:3, :].set(
        jnp.sum(w1_khj, axis=1).T)
    w2t = W2.T.astype(jnp.bfloat16)
    bn = jnp.stack([g1, be1, g2, be2], axis=-1)

    nc = N // _NCHUNK
    tiles_per_chunk = nc // _TN
    outs = []
    for c in range(_NCHUNK):
        idx_c = idx[:, c * nc:(c + 1) * nc, :].transpose(0, 2, 1)
        feats_c = _sc_gather_feats(
            xt, idx_c.reshape(1, B * K * nc)).reshape(B, K, nc, C)
        outs.append(_mlp_call(
            pts_p, n0, n1, n2, feats_c, bn, w1c, w10, w11, w12,
            b1.reshape(1, HID), w2t, b2.reshape(1, OUTD),
            c * tiles_per_chunk, tiles_per_chunk))
    out = jnp.concatenate(outs, axis=1)
    return (out, points, indices)


# speedup vs baseline: 31.3235x; 1.0314x over previous
import dataclasses
import functools

import jax
import jax.numpy as jnp
from jax.experimental import pallas as pl
from jax.experimental.pallas import tpu as pltpu
from jax.experimental.pallas import tpu_sc as plsc

_P = 8
_TN = 200
_GW = 128
_NW = 32
_VEC = 16
_NCHUNK = 5


def _sc_gather_feats(xt, idx):
    M = idx.shape[1]
    C = xt.shape[1]
    mesh = plsc.VectorSubcoreMesh(core_axis_name="c", subcore_axis_name="s")

    @functools.partial(
        pl.kernel,
        out_type=jax.ShapeDtypeStruct((M, C), xt.dtype),
        mesh=mesh,
    )
    def k(x_hbm, i_hbm, o_hbm):
        def body(i_vmem, o_vmem):
            pltpu.sync_copy(x_hbm.at[i_vmem.at[0]], o_vmem)

        pltpu.emit_pipeline(
            body,
            grid=(M // _GW,),
            in_specs=[pl.BlockSpec((1, _GW), lambda i: (0, i))],
            out_specs=[pl.BlockSpec((_GW, C), lambda i: (i, 0))],
            core_axis_name=("c", "s"),
            dimension_semantics=(pltpu.PARALLEL,),
        )(i_hbm, o_hbm)

    return k(xt, idx)


def _sc_gather_coords(px, py, pz, idx):
    R = px.shape[0]
    M = idx.shape[0]
    per_w = M // _NW
    CH = 2000
    mesh = plsc.VectorSubcoreMesh(core_axis_name="c", subcore_axis_name="s")
    cp = pltpu.CompilerParams()
    if "needs_layout_passes" in pltpu.CompilerParams.__dataclass_fields__:
        cp = dataclasses.replace(cp, needs_layout_passes=False)

    @functools.partial(
        pl.kernel,
        out_type=tuple(jax.ShapeDtypeStruct((M,), jnp.float32)
                       for _ in range(3)),
        mesh=mesh,
        compiler_params=cp,
        scratch_types=[
            pltpu.VMEM((R,), jnp.float32),
            pltpu.VMEM((R,), jnp.float32),
            pltpu.VMEM((R,), jnp.float32),
            pltpu.VMEM((CH,), jnp.int32),
            pltpu.VMEM((CH,), jnp.float32),
            pltpu.VMEM((CH,), jnp.float32),
            pltpu.VMEM((CH,), jnp.float32),
        ],
    )
    def k(px_hbm, py_hbm, pz_hbm, i_hbm, o0_hbm, o1_hbm, o2_hbm,
          tx_v, ty_v, tz_v, i_v, o0_v, o1_v, o2_v):
        wid = jax.lax.axis_index("s") * 2 + jax.lax.axis_index("c")
        base = wid * per_w
        pltpu.sync_copy(px_hbm, tx_v)
        pltpu.sync_copy(py_hbm, ty_v)
        pltpu.sync_copy(pz_hbm, tz_v)

        @pl.loop(0, per_w, step=CH)
        def _chunk(c0):
            pltpu.sync_copy(i_hbm.at[pl.ds(base + c0, CH)], i_v)

            @pl.loop(0, CH, step=_VEC)
            def _vec(t):
                iv = i_v[pl.ds(t, _VEC)]
                o0_v[pl.ds(t, _VEC)] = plsc.load_gather(tx_v, [iv])
                o1_v[pl.ds(t, _VEC)] = plsc.load_gather(ty_v, [iv])
                o2_v[pl.ds(t, _VEC)] = plsc.load_gather(tz_v, [iv])

            pltpu.sync_copy(o0_v, o0_hbm.at[pl.ds(base + c0, CH)])
            pltpu.sync_copy(o1_v, o1_hbm.at[pl.ds(base + c0, CH)])
            pltpu.sync_copy(o2_v, o2_hbm.at[pl.ds(base + c0, CH)])

    return k(px, py, pz, idx)


def _bn_relu(y, g, be):
    cnt = y.shape[0] * y.shape[2]
    s = jnp.sum(y, axis=2, keepdims=True)
    ss = jnp.sum(y * y, axis=2, keepdims=True)
    mean = jnp.sum(s, axis=0, keepdims=True) / cnt
    ex2 = jnp.sum(ss, axis=0, keepdims=True) / cnt
    var = ex2 - mean * mean
    inv = jax.lax.rsqrt(var + 1e-5)
    return jnp.maximum((y - mean) * (inv * g[None]) + be[None], 0.0)


def _mlp_body(m_ref, feat_ref, bn_ref, w1_ref, b1_ref, w2_ref, b2_ref,
              out_ref):
    B, TN, D1 = m_ref.shape
    HID = w1_ref.shape[1]
    OUTD = w2_ref.shape[1]
    C = out_ref.shape[2]
    K = OUTD // C

    dot = functools.partial(jnp.dot, preferred_element_type=jnp.float32)
    y = dot(m_ref[...].reshape(B * TN, D1), w1_ref[...]) + b1_ref[...]
    y = _bn_relu(y.reshape(B, TN, HID), bn_ref[:, 0:1], bn_ref[:, 1:2])
    z = (dot(y.reshape(B * TN, HID).astype(w2_ref.dtype), w2_ref[...])
         + b2_ref[...])
    z = _bn_relu(z.reshape(B, TN, OUTD), bn_ref[:, 2:3], bn_ref[:, 3:4])

    f = feat_ref[...]
    acc = z[:, :, 0:C] * f[:, 0]
    for k in range(1, K):
        acc = acc + z[:, :, k * C:(k + 1) * C] * f[:, k]
    out_ref[...] = acc


def _mlp_call(m, feats, bn, w1, b1, w2t, b2, tile0, ntiles):
    B, _, D1 = m.shape
    K = feats.shape[1]
    C = feats.shape[3]
    HID = w2t.shape[0]
    OUTD = w2t.shape[1]
    return pl.pallas_call(
        _mlp_body,
        grid=(ntiles,),
        in_specs=[
            pl.BlockSpec((B, _TN, D1), lambda i: (0, i + tile0, 0)),
            pl.BlockSpec((B, K, _TN, C), lambda i: (0, 0, i, 0)),
            pl.BlockSpec((_TN, 4), lambda i: (i + tile0, 0)),
            pl.BlockSpec((D1, HID), lambda i: (0, 0)),
            pl.BlockSpec((1, HID), lambda i: (0, 0)),
            pl.BlockSpec((HID, OUTD), lambda i: (0, 0)),
            pl.BlockSpec((1, OUTD), lambda i: (0, 0)),
        ],
        out_specs=pl.BlockSpec((B, _TN, C), lambda i: (0, i, 0)),
        out_shape=jax.ShapeDtypeStruct((B, ntiles * _TN, C), jnp.float32),
    )(m, feats, bn, w1, b1, w2t, b2)


def kernel(x, points, indices, W1, b1, g1, be1, W2, b2, g2, be2):
    B, N, C = x.shape
    K = indices.shape[2]
    HID = W1.shape[0]
    OUTD = W2.shape[0]

    xt = x.reshape(B * N, C)
    pf = points.reshape(B * N, 3)
    idx = (indices.astype(jnp.int32)
           + (jnp.arange(B, dtype=jnp.int32) * N)[:, None, None])
    n0, n1, n2 = _sc_gather_coords(pf[:, 0], pf[:, 1], pf[:, 2],
                                   idx.reshape(B * N * K))

    pts_p = jnp.pad(points, ((0, 0), (0, 0), (0, _P - 3)))
    m = jnp.concatenate(
        [pts_p, n0.reshape(B, N, K), n1.reshape(B, N, K),
         n2.reshape(B, N, K)], axis=2)
    w1_khj = W1.reshape(HID, K, 3)
    w1c = jnp.zeros((_P, HID), W1.dtype).at[:3, :].set(
        jnp.sum(w1_khj, axis=1).T)
    w1 = jnp.concatenate(
        [w1c, -w1_khj[:, :, 0].T, -w1_khj[:, :, 1].T, -w1_khj[:, :, 2].T],
        axis=0)
    w2t = W2.T.astype(jnp.bfloat16)
    bn = jnp.stack([g1, be1, g2, be2], axis=-1)

    nc = N // _NCHUNK
    tiles_per_chunk = nc // _TN
    outs = []
    for c in range(_NCHUNK):
        idx_c = idx[:, c * nc:(c + 1) * nc, :].transpose(0, 2, 1)
        feats_c = _sc_gather_feats(
            xt, idx_c.reshape(1, B * K * nc)).reshape(B, K, nc, C)
        outs.append(_mlp_call(
            m, feats_c, bn, w1, b1.reshape(1, HID), w2t,
            b2.reshape(1, OUTD), c * tiles_per_chunk, tiles_per_chunk))
    out = jnp.concatenate(outs, axis=1)
    return (out, points, indices)


# speedup vs baseline: 31.8980x; 1.0183x over previous
import dataclasses
import functools

import jax
import jax.numpy as jnp
from jax.experimental import pallas as pl
from jax.experimental.pallas import tpu as pltpu
from jax.experimental.pallas import tpu_sc as plsc

_P = 8
_TN = 400
_GW = 128
_NW = 32
_VEC = 16
_NCHUNK = 5


def _sc_gather_feats(xt, idx):
    M = idx.shape[1]
    C = xt.shape[1]
    mesh = plsc.VectorSubcoreMesh(core_axis_name="c", subcore_axis_name="s")

    @functools.partial(
        pl.kernel,
        out_type=jax.ShapeDtypeStruct((M, C), xt.dtype),
        mesh=mesh,
    )
    def k(x_hbm, i_hbm, o_hbm):
        def body(i_vmem, o_vmem):
            pltpu.sync_copy(x_hbm.at[i_vmem.at[0]], o_vmem)

        pltpu.emit_pipeline(
            body,
            grid=(M // _GW,),
            in_specs=[pl.BlockSpec((1, _GW), lambda i: (0, i))],
            out_specs=[pl.BlockSpec((_GW, C), lambda i: (i, 0))],
            core_axis_name=("c", "s"),
            dimension_semantics=(pltpu.PARALLEL,),
        )(i_hbm, o_hbm)

    return k(xt, idx)


def _sc_gather_coords(px, py, pz, idx):
    R = px.shape[0]
    M = idx.shape[0]
    per_w = M // _NW
    CH = 2000
    mesh = plsc.VectorSubcoreMesh(core_axis_name="c", subcore_axis_name="s")
    cp = pltpu.CompilerParams()
    if "needs_layout_passes" in pltpu.CompilerParams.__dataclass_fields__:
        cp = dataclasses.replace(cp, needs_layout_passes=False)

    @functools.partial(
        pl.kernel,
        out_type=tuple(jax.ShapeDtypeStruct((M,), jnp.float32)
                       for _ in range(3)),
        mesh=mesh,
        compiler_params=cp,
        scratch_types=[
            pltpu.VMEM((R,), jnp.float32),
            pltpu.VMEM((R,), jnp.float32),
            pltpu.VMEM((R,), jnp.float32),
            pltpu.VMEM((CH,), jnp.int32),
            pltpu.VMEM((CH,), jnp.float32),
            pltpu.VMEM((CH,), jnp.float32),
            pltpu.VMEM((CH,), jnp.float32),
        ],
    )
    def k(px_hbm, py_hbm, pz_hbm, i_hbm, o0_hbm, o1_hbm, o2_hbm,
          tx_v, ty_v, tz_v, i_v, o0_v, o1_v, o2_v):
        wid = jax.lax.axis_index("s") * 2 + jax.lax.axis_index("c")
        base = wid * per_w
        pltpu.sync_copy(px_hbm, tx_v)
        pltpu.sync_copy(py_hbm, ty_v)
        pltpu.sync_copy(pz_hbm, tz_v)

        @pl.loop(0, per_w, step=CH)
        def _chunk(c0):
            pltpu.sync_copy(i_hbm.at[pl.ds(base + c0, CH)], i_v)

            @pl.loop(0, CH, step=_VEC)
            def _vec(t):
                iv = i_v[pl.ds(t, _VEC)]
                o0_v[pl.ds(t, _VEC)] = plsc.load_gather(tx_v, [iv])
                o1_v[pl.ds(t, _VEC)] = plsc.load_gather(ty_v, [iv])
                o2_v[pl.ds(t, _VEC)] = plsc.load_gather(tz_v, [iv])

            pltpu.sync_copy(o0_v, o0_hbm.at[pl.ds(base + c0, CH)])
            pltpu.sync_copy(o1_v, o1_hbm.at[pl.ds(base + c0, CH)])
            pltpu.sync_copy(o2_v, o2_hbm.at[pl.ds(base + c0, CH)])

    return k(px, py, pz, idx)


def _bn_relu(y, g, be):
    cnt = y.shape[0] * y.shape[2]
    s = jnp.sum(y, axis=2, keepdims=True)
    ss = jnp.sum(y * y, axis=2, keepdims=True)
    mean = jnp.sum(s, axis=0, keepdims=True) / cnt
    ex2 = jnp.sum(ss, axis=0, keepdims=True) / cnt
    var = ex2 - mean * mean
    inv = jax.lax.rsqrt(var + 1e-5)
    return jnp.maximum((y - mean) * (inv * g[None]) + be[None], 0.0)


def _mlp_body(m_ref, feat_ref, bn_ref, w1_ref, b1_ref, w2_ref, b2_ref,
              out_ref):
    B, TN, D1 = m_ref.shape
    HID = w1_ref.shape[1]
    OUTD = w2_ref.shape[1]
    C = out_ref.shape[2]
    K = OUTD // C

    dot = functools.partial(jnp.dot, preferred_element_type=jnp.float32)
    y = dot(m_ref[...].reshape(B * TN, D1), w1_ref[...]) + b1_ref[...]
    y = _bn_relu(y.reshape(B, TN, HID), bn_ref[:, 0:1], bn_ref[:, 1:2])
    z = (dot(y.reshape(B * TN, HID).astype(w2_ref.dtype), w2_ref[...])
         + b2_ref[...])
    z = _bn_relu(z.reshape(B, TN, OUTD), bn_ref[:, 2:3], bn_ref[:, 3:4])

    f = feat_ref[...]
    acc = z[:, :, 0:C] * f[:, 0]
    for k in range(1, K):
        acc = acc + z[:, :, k * C:(k + 1) * C] * f[:, k]
    out_ref[...] = acc


def _mlp_call(m, feats, bn, w1, b1, w2t, b2, tile0, ntiles):
    B, _, D1 = m.shape
    K = feats.shape[1]
    C = feats.shape[3]
    HID = w2t.shape[0]
    OUTD = w2t.shape[1]
    return pl.pallas_call(
        _mlp_body,
        grid=(ntiles,),
        in_specs=[
            pl.BlockSpec((B, _TN, D1), lambda i: (0, i + tile0, 0)),
            pl.BlockSpec((B, K, _TN, C), lambda i: (0, 0, i, 0)),
            pl.BlockSpec((_TN, 4), lambda i: (i + tile0, 0)),
            pl.BlockSpec((D1, HID), lambda i: (0, 0)),
            pl.BlockSpec((1, HID), lambda i: (0, 0)),
            pl.BlockSpec((HID, OUTD), lambda i: (0, 0)),
            pl.BlockSpec((1, OUTD), lambda i: (0, 0)),
        ],
        out_specs=pl.BlockSpec((B, _TN, C), lambda i: (0, i, 0)),
        out_shape=jax.ShapeDtypeStruct((B, ntiles * _TN, C), jnp.float32),
    )(m, feats, bn, w1, b1, w2t, b2)


def kernel(x, points, indices, W1, b1, g1, be1, W2, b2, g2, be2):
    B, N, C = x.shape
    K = indices.shape[2]
    HID = W1.shape[0]
    OUTD = W2.shape[0]

    xt = x.reshape(B * N, C)
    pf = points.reshape(B * N, 3)
    idx = (indices.astype(jnp.int32)
           + (jnp.arange(B, dtype=jnp.int32) * N)[:, None, None])
    n0, n1, n2 = _sc_gather_coords(pf[:, 0], pf[:, 1], pf[:, 2],
                                   idx.reshape(B * N * K))

    pts_p = jnp.pad(points, ((0, 0), (0, 0), (0, _P - 3)))
    m = jnp.concatenate(
        [pts_p, n0.reshape(B, N, K), n1.reshape(B, N, K),
         n2.reshape(B, N, K)], axis=2)
    w1_khj = W1.reshape(HID, K, 3)
    w1c = jnp.zeros((_P, HID), W1.dtype).at[:3, :].set(
        jnp.sum(w1_khj, axis=1).T)
    w1 = jnp.concatenate(
        [w1c, -w1_khj[:, :, 0].T, -w1_khj[:, :, 1].T, -w1_khj[:, :, 2].T],
        axis=0)
    w2t = W2.T.astype(jnp.bfloat16)
    bn = jnp.stack([g1, be1, g2, be2], axis=-1)

    nc = N // _NCHUNK
    tiles_per_chunk = nc // _TN
    outs = []
    for c in range(_NCHUNK):
        idx_c = idx[:, c * nc:(c + 1) * nc, :].transpose(0, 2, 1)
        feats_c = _sc_gather_feats(
            xt, idx_c.reshape(1, B * K * nc)).reshape(B, K, nc, C)
        outs.append(_mlp_call(
            m, feats_c, bn, w1, b1.reshape(1, HID), w2t,
            b2.reshape(1, OUTD), c * tiles_per_chunk, tiles_per_chunk))
    out = jnp.concatenate(outs, axis=1)
    return (out, points, indices)
